# baseline (speedup 1.0000x reference)
import functools

import jax
import jax.numpy as jnp
from jax import lax
from jax.experimental import pallas as pl
from jax.experimental.pallas import tpu as pltpu
from jax.experimental.pallas import tpu_sc as plsc

N_NODES = 100000
N_EDGES = 6400000

NC = 2
NS = 16

CHUNK = 125
ROWS = 51200
ROWS_W = ROWS // (NC * NS)
SROWS_D = 200
NSTAGE_D = ROWS_W // SROWS_D
QLAG = 8
SROWS = 40
NSTAGE = ROWS_W // SROWS
RING = 8
PD = 4
NGROUP = SROWS // RING

NPAD = 100352
SLICE = NPAD // NS
ZROWS = 128

_mesh = plsc.VectorSubcoreMesh(
    core_axis_name="c", subcore_axis_name="s", num_cores=NC, num_subcores=NS)


def _deg_body(dst_hbm, deg_out, deg_sh, ids, ones_v, zbuf, qsem):
    c = lax.axis_index("c")
    s = lax.axis_index("s")

    def _zb(i, _):
        zbuf[pl.ds(i * 16, 16)] = jnp.zeros((16,), jnp.float32)
        return 0
    lax.fori_loop(0, SLICE // 16, _zb, 0)

    def _ob(i, _):
        ones_v[pl.ds(i * 16, 16)] = jnp.ones((16,), jnp.float32)
        return 0
    lax.fori_loop(0, 8, _ob, 0)

    pltpu.sync_copy(zbuf, deg_sh.at[pl.ds(s * SLICE, SLICE)])
    plsc.subcore_barrier()

    row0 = (c * NS + s) * ROWS_W

    def _stage(st, _):
        pltpu.sync_copy(dst_hbm.at[pl.ds(row0 + st * SROWS_D, SROWS_D)], ids)

        ones_c = ones_v.at[pl.ds(0, CHUNK)]

        def _wait_one():
            pltpu.make_async_copy(
                ones_c, deg_sh.at[ids.at[0]], qsem).wait()

        for k in range(QLAG):
            pltpu.async_copy(ones_c, deg_sh.at[ids.at[k]], qsem, add=True)

        def _chunk(j, _):
            _wait_one()
            pltpu.async_copy(
                ones_c, deg_sh.at[ids.at[j + QLAG]], qsem, add=True)
            return 0
        lax.fori_loop(0, SROWS_D - QLAG, _chunk, 0)
        for k in range(QLAG):
            _wait_one()
        return 0
    lax.fori_loop(0, NSTAGE_D, _stage, 0)

    plsc.subcore_barrier()
    pltpu.sync_copy(deg_sh.at[pl.ds(s * SLICE, SLICE)], zbuf)
    pltpu.sync_copy(zbuf, deg_out.at[pl.ds(c * NPAD + s * SLICE, SLICE)])


_deg_call = functools.partial(
    pl.kernel,
    out_type=jax.ShapeDtypeStruct((NC * NPAD,), jnp.float32),
    mesh=_mesh,
    scratch_types=[
        pltpu.VMEM_SHARED((NPAD,), jnp.float32),
        pltpu.VMEM((SROWS_D, CHUNK), jnp.int32),
        pltpu.VMEM((128,), jnp.float32),
        pltpu.VMEM((SLICE,), jnp.float32),
        pltpu.SemaphoreType.DMA,
    ],
    compiler_params=pltpu.CompilerParams(use_tc_tiling_on_sc=False),
)(_deg_body)


def _msg_body(y_hbm, src_hbm, dst_hbm, acc_out, acc_sh, sid, did, rows, zbuf,
              *sems):
    gsems = sems[:RING]
    ssems = sems[RING:]
    c = lax.axis_index("c")
    s = lax.axis_index("s")

    def _zb(i, _):
        zbuf[i, :] = jnp.zeros((16,), jnp.float32)
        return 0
    lax.fori_loop(0, ZROWS, _zb, 0)
    for k in range(SLICE // ZROWS):
        pltpu.sync_copy(zbuf, acc_sh.at[pl.ds(s * SLICE + k * ZROWS, ZROWS)])
    plsc.subcore_barrier()

    row0 = (c * NS + s) * ROWS_W

    def _gather(j, b):
        pltpu.async_copy(y_hbm.at[sid.at[j]], rows.at[b], gsems[b])

    def _gather_wait(j, b):
        pltpu.make_async_copy(y_hbm.at[sid.at[j]], rows.at[b],
                              gsems[b]).wait()

    def _scatter(j, b):
        pltpu.async_copy(rows.at[b], acc_sh.at[did.at[j]], ssems[b],
                         add=True)

    def _scatter_wait(j, b):
        pltpu.make_async_copy(rows.at[b], acc_sh.at[did.at[j]],
                              ssems[b]).wait()

    def _stage(st, _):
        base = row0 + st * SROWS
        pltpu.sync_copy(src_hbm.at[pl.ds(base, SROWS)], sid)
        pltpu.sync_copy(dst_hbm.at[pl.ds(base, SROWS)], did)

        for b in range(PD):
            _gather(b, b)
        for b in range(RING):
            _gather_wait(b, b)
            _scatter(b, b)
            bp = (b + PD) % RING
            if b + PD >= RING:
                _scatter_wait(b + PD - RING, bp)
            _gather(b + PD, bp)

        def _group(g, _):
            j0 = g * RING
            for b in range(RING):
                j = j0 + b
                _gather_wait(j, b)
                _scatter(j, b)
                bp = (b + PD) % RING
                _scatter_wait(j + PD - RING, bp)
                _gather(j + PD, bp)
            return 0
        lax.fori_loop(1, NGROUP - 1, _group, 0)

        j0 = (NGROUP - 1) * RING
        for b in range(RING):
            j = j0 + b
            _gather_wait(j, b)
            _scatter(j, b)
            if b < PD:
                bp = (b + PD) % RING
                _scatter_wait(j + PD - RING, bp)
                _gather(j + PD, bp)
        for b in range(RING):
            _scatter_wait(j0 + b, b)
        return 0
    lax.fori_loop(0, NSTAGE, _stage, 0)

    plsc.subcore_barrier()
    for k in range(SLICE // ZROWS):
        off = s * SLICE + k * ZROWS
        pltpu.sync_copy(acc_sh.at[pl.ds(off, ZROWS)], zbuf)
        pltpu.sync_copy(zbuf, acc_out.at[c, pl.ds(off, ZROWS)])


_msg_call = functools.partial(
    pl.kernel,
    out_type=jax.ShapeDtypeStruct((NC, NPAD, 16), jnp.float32),
    mesh=_mesh,
    scratch_types=[
        pltpu.VMEM_SHARED((NPAD, 16), jnp.float32),
        pltpu.VMEM((SROWS, CHUNK), jnp.int32),
        pltpu.VMEM((SROWS, CHUNK), jnp.int32),
        pltpu.VMEM((RING, CHUNK, 16), jnp.float32),
        pltpu.VMEM((ZROWS, 16), jnp.float32),
    ] + [pltpu.SemaphoreType.DMA] * (2 * RING),
    compiler_params=pltpu.CompilerParams(use_tc_tiling_on_sc=False),
)(_msg_body)


def _y_kernel(x_ref, w_ref, degs_ref, y_ref):
    deg = degs_ref[:, 0] + degs_ref[:, 1] + 1.0
    dinv = lax.rsqrt(deg)
    xw = jnp.dot(x_ref[...], w_ref[...], preferred_element_type=jnp.float32)
    y_ref[...] = xw * dinv[:, None]


def _combine_kernel(acc_ref, y_ref, degs_ref, b_ref, hid_ref):
    deg = degs_ref[:, 0] + degs_ref[:, 1] + 1.0
    dinv = lax.rsqrt(deg)
    tot = acc_ref[0] + acc_ref[1] + y_ref[...]
    hid_ref[...] = tot * dinv[:, None] + b_ref[...][None, :]


def _lin_kernel(h_ref, wl_ref, bl_ref, out_ref):
    h = jnp.maximum(h_ref[...], 0.0)
    out_ref[...] = (
        jnp.dot(h, wl_ref[...], preferred_element_type=jnp.float32)
        + bl_ref[...][None, :])


def kernel(x, edge_index, W, b, Wl, bl):
    src2d = edge_index[0].reshape(ROWS, CHUNK)
    dst2d = edge_index[1].reshape(ROWS, CHUNK)

    degs_t = _deg_call(dst2d).reshape(NC, NPAD).T

    BR = 4000
    y = pl.pallas_call(
        _y_kernel,
        grid=(N_NODES // BR,),
        in_specs=[
            pl.BlockSpec((BR, 8), lambda i: (i, 0)),
            pl.BlockSpec((8, 16), lambda i: (0, 0)),
            pl.BlockSpec((BR, NC), lambda i: (i, 0)),
        ],
        out_specs=pl.BlockSpec((BR, 16), lambda i: (i, 0)),
        out_shape=jax.ShapeDtypeStruct((N_NODES, 16), jnp.float32),
    )(x, W, degs_t)

    accs = _msg_call(y, src2d, dst2d)

    hidden = pl.pallas_call(
        _combine_kernel,
        grid=(N_NODES // BR,),
        in_specs=[
            pl.BlockSpec((NC, BR, 16), lambda i: (0, i, 0)),
            pl.BlockSpec((BR, 16), lambda i: (i, 0)),
            pl.BlockSpec((BR, NC), lambda i: (i, 0)),
            pl.BlockSpec((16,), lambda i: (0,)),
        ],
        out_specs=pl.BlockSpec((BR, 16), lambda i: (i, 0)),
        out_shape=jax.ShapeDtypeStruct((N_NODES, 16), jnp.float32),
    )(accs, y, degs_t, b)

    h80 = hidden.reshape(N_NODES // 5, 80)
    BL = 800
    out2d = pl.pallas_call(
        _lin_kernel,
        grid=(N_NODES // 5 // BL,),
        in_specs=[
            pl.BlockSpec((BL, 80), lambda i: (i, 0)),
            pl.BlockSpec((80, 445), lambda i: (0, 0)),
            pl.BlockSpec((445,), lambda i: (0,)),
        ],
        out_specs=pl.BlockSpec((BL, 445), lambda i: (i, 0)),
        out_shape=jax.ShapeDtypeStruct((N_NODES // 5, 445), jnp.float32),
    )(h80, Wl, bl)

    return out2d.reshape(-1, 89, 5), hidden


# speedup vs baseline: 1.5557x; 1.5557x over previous
import functools

import jax
import jax.numpy as jnp
from jax import lax
from jax.experimental import pallas as pl
from jax.experimental.pallas import tpu as pltpu
from jax.experimental.pallas import tpu_sc as plsc

N_NODES = 100000
N_EDGES = 6400000

NC = 2
NS = 16

CHUNK = 128
ROWS = N_EDGES // CHUNK
ROWS_W = 1560
TAIL0 = 32 * ROWS_W
SROWS_D = 120
NSTAGE_D = ROWS_W // SROWS_D
QLAG = 8
SROWS = 40
NSTAGE = ROWS_W // SROWS
RING = 8
PD = 4
NGROUP = SROWS // RING

NPAD = 100352
SLICE = NPAD // NS
ZROWS = 128

_mesh = plsc.VectorSubcoreMesh(
    core_axis_name="c", subcore_axis_name="s", num_cores=NC, num_subcores=NS)


def _deg_body(ei_hbm, deg_out, deg_sh, ids, ones_v, zbuf, qsem):
    c = lax.axis_index("c")
    s = lax.axis_index("s")

    def _zb(i, _):
        zbuf[pl.ds(i * 16, 16)] = jnp.zeros((16,), jnp.float32)
        return 0
    lax.fori_loop(0, SLICE // 16, _zb, 0)

    def _ob(i, _):
        ones_v[pl.ds(i * 16, 16)] = jnp.ones((16,), jnp.float32)
        return 0
    lax.fori_loop(0, 8, _ob, 0)

    pltpu.sync_copy(zbuf, deg_sh.at[pl.ds(s * SLICE, SLICE)])
    plsc.subcore_barrier()

    wid = c * NS + s
    row0 = wid * ROWS_W

    def _wait_one():
        pltpu.make_async_copy(
            ones_v, deg_sh.at[ids.at[0]], qsem).wait()

    def _stage(st, _):
        pltpu.sync_copy(
            ei_hbm.at[1, pl.ds(row0 + st * SROWS_D, SROWS_D)], ids)

        for k in range(QLAG):
            pltpu.async_copy(ones_v, deg_sh.at[ids.at[k]], qsem, add=True)

        def _chunk(j, _):
            _wait_one()
            pltpu.async_copy(
                ones_v, deg_sh.at[ids.at[j + QLAG]], qsem, add=True)
            return 0
        lax.fori_loop(0, SROWS_D - QLAG, _chunk, 0)
        for k in range(QLAG):
            _wait_one()
        return 0
    lax.fori_loop(0, NSTAGE_D, _stage, 0)

    @pl.when(wid < 10)
    def _tail():
        pltpu.sync_copy(ei_hbm.at[1, pl.ds(TAIL0 + wid * 8, 8)],
                        ids.at[pl.ds(0, 8)])
        for k in range(8):
            pltpu.async_copy(ones_v, deg_sh.at[ids.at[k]], qsem, add=True)
        for k in range(8):
            _wait_one()

    plsc.subcore_barrier()
    pltpu.sync_copy(deg_sh.at[pl.ds(s * SLICE, SLICE)], zbuf)
    pltpu.sync_copy(zbuf, deg_out.at[pl.ds(c * NPAD + s * SLICE, SLICE)])


_deg_call = functools.partial(
    pl.kernel,
    out_type=jax.ShapeDtypeStruct((NC * NPAD,), jnp.float32),
    mesh=_mesh,
    scratch_types=[
        pltpu.VMEM_SHARED((NPAD,), jnp.float32),
        pltpu.VMEM((SROWS_D, CHUNK), jnp.int32),
        pltpu.VMEM((CHUNK,), jnp.float32),
        pltpu.VMEM((SLICE,), jnp.float32),
        pltpu.SemaphoreType.DMA,
    ],
    compiler_params=pltpu.CompilerParams(use_tc_tiling_on_sc=False),
)(_deg_body)


def _msg_body(y_hbm, ei_hbm, acc_out, acc_sh, sid, did, rows, zbuf,
              *sems):
    gsems = sems[:RING]
    ssems = sems[RING:]
    c = lax.axis_index("c")
    s = lax.axis_index("s")

    def _zb(i, _):
        zbuf[i, :] = jnp.zeros((16,), jnp.float32)
        return 0
    lax.fori_loop(0, ZROWS, _zb, 0)
    for k in range(SLICE // ZROWS):
        pltpu.sync_copy(zbuf, acc_sh.at[pl.ds(s * SLICE + k * ZROWS, ZROWS)])
    plsc.subcore_barrier()

    wid = c * NS + s
    row0 = wid * ROWS_W

    def _gather(j, b):
        pltpu.async_copy(y_hbm.at[sid.at[j]], rows.at[b], gsems[b])

    def _gather_wait(j, b):
        pltpu.make_async_copy(y_hbm.at[sid.at[j]], rows.at[b],
                              gsems[b]).wait()

    def _scatter(j, b):
        pltpu.async_copy(rows.at[b], acc_sh.at[did.at[j]], ssems[b],
                         add=True)

    def _scatter_wait(j, b):
        pltpu.make_async_copy(rows.at[b], acc_sh.at[did.at[j]],
                              ssems[b]).wait()

    def _stage(st, _):
        base = row0 + st * SROWS
        pltpu.sync_copy(ei_hbm.at[0, pl.ds(base, SROWS)], sid)
        pltpu.sync_copy(ei_hbm.at[1, pl.ds(base, SROWS)], did)

        for b in range(PD):
            _gather(b, b)
        for b in range(RING):
            _gather_wait(b, b)
            _scatter(b, b)
            bp = (b + PD) % RING
            if b + PD >= RING:
                _scatter_wait(b + PD - RING, bp)
            _gather(b + PD, bp)

        def _group(g, _):
            j0 = g * RING
            for b in range(RING):
                j = j0 + b
                _gather_wait(j, b)
                _scatter(j, b)
                bp = (b + PD) % RING
                _scatter_wait(j + PD - RING, bp)
                _gather(j + PD, bp)
            return 0
        lax.fori_loop(1, NGROUP - 1, _group, 0)

        j0 = (NGROUP - 1) * RING
        for b in range(RING):
            j = j0 + b
            _gather_wait(j, b)
            _scatter(j, b)
            if b < PD:
                bp = (b + PD) % RING
                _scatter_wait(j + PD - RING, bp)
                _gather(j + PD, bp)
        for b in range(RING):
            _scatter_wait(j0 + b, b)
        return 0
    lax.fori_loop(0, NSTAGE, _stage, 0)

    @pl.when(wid < 10)
    def _tail():
        pltpu.sync_copy(ei_hbm.at[0, pl.ds(TAIL0 + wid * 8, 8)],
                        sid.at[pl.ds(0, 8)])
        pltpu.sync_copy(ei_hbm.at[1, pl.ds(TAIL0 + wid * 8, 8)],
                        did.at[pl.ds(0, 8)])
        for k in range(8):
            b = k % RING
            pltpu.async_copy(y_hbm.at[sid.at[k]], rows.at[b],
                             gsems[b]).wait()
            pltpu.sync_copy(rows.at[b], acc_sh.at[did.at[k]], add=True)

    plsc.subcore_barrier()
    for k in range(SLICE // ZROWS):
        off = s * SLICE + k * ZROWS
        pltpu.sync_copy(acc_sh.at[pl.ds(off, ZROWS)], zbuf)
        pltpu.sync_copy(zbuf, acc_out.at[c, pl.ds(off, ZROWS)])


_msg_call = functools.partial(
    pl.kernel,
    out_type=jax.ShapeDtypeStruct((NC, NPAD, 16), jnp.float32),
    mesh=_mesh,
    scratch_types=[
        pltpu.VMEM_SHARED((NPAD, 16), jnp.float32),
        pltpu.VMEM((SROWS, CHUNK), jnp.int32),
        pltpu.VMEM((SROWS, CHUNK), jnp.int32),
        pltpu.VMEM((RING, CHUNK, 16), jnp.float32),
        pltpu.VMEM((ZROWS, 16), jnp.float32),
    ] + [pltpu.SemaphoreType.DMA] * (2 * RING),
    compiler_params=pltpu.CompilerParams(use_tc_tiling_on_sc=False),
)(_msg_body)


def _y_kernel(x_ref, w_ref, degs_ref, y_ref):
    deg = degs_ref[:, 0] + degs_ref[:, 1] + 1.0
    dinv = lax.rsqrt(deg)
    xw = jnp.dot(x_ref[...], w_ref[...], preferred_element_type=jnp.float32)
    y_ref[...] = xw * dinv[:, None]


def _combine_kernel(acc_ref, y_ref, degs_ref, b_ref, hid_ref):
    deg = degs_ref[:, 0] + degs_ref[:, 1] + 1.0
    dinv = lax.rsqrt(deg)
    tot = acc_ref[0] + acc_ref[1] + y_ref[...]
    hid_ref[...] = tot * dinv[:, None] + b_ref[...][None, :]


def _lin_kernel(h_ref, wl_ref, bl_ref, out_ref):
    h = jnp.maximum(h_ref[...], 0.0)
    out_ref[...] = (
        jnp.dot(h, wl_ref[...], preferred_element_type=jnp.float32)
        + bl_ref[...][None, :])


def kernel(x, edge_index, W, b, Wl, bl):
    ei3 = edge_index.reshape(2, ROWS, CHUNK)

    degs_t = _deg_call(ei3).reshape(NC, NPAD).T

    BR = 4000
    y = pl.pallas_call(
        _y_kernel,
        grid=(N_NODES // BR,),
        in_specs=[
            pl.BlockSpec((BR, 8), lambda i: (i, 0)),
            pl.BlockSpec((8, 16), lambda i: (0, 0)),
            pl.BlockSpec((BR, NC), lambda i: (i, 0)),
        ],
        out_specs=pl.BlockSpec((BR, 16), lambda i: (i, 0)),
        out_shape=jax.ShapeDtypeStruct((N_NODES, 16), jnp.float32),
    )(x, W, degs_t)

    accs = _msg_call(y, ei3)

    hidden = pl.pallas_call(
        _combine_kernel,
        grid=(N_NODES // BR,),
        in_specs=[
            pl.BlockSpec((NC, BR, 16), lambda i: (0, i, 0)),
            pl.BlockSpec((BR, 16), lambda i: (i, 0)),
            pl.BlockSpec((BR, NC), lambda i: (i, 0)),
            pl.BlockSpec((16,), lambda i: (0,)),
        ],
        out_specs=pl.BlockSpec((BR, 16), lambda i: (i, 0)),
        out_shape=jax.ShapeDtypeStruct((N_NODES, 16), jnp.float32),
    )(accs, y, degs_t, b)

    h80 = hidden.reshape(N_NODES // 5, 80)
    BL = 800
    out2d = pl.pallas_call(
        _lin_kernel,
        grid=(N_NODES // 5 // BL,),
        in_specs=[
            pl.BlockSpec((BL, 80), lambda i: (i, 0)),
            pl.BlockSpec((80, 445), lambda i: (0, 0)),
            pl.BlockSpec((445,), lambda i: (0,)),
        ],
        out_specs=pl.BlockSpec((BL, 445), lambda i: (i, 0)),
        out_shape=jax.ShapeDtypeStruct((N_NODES // 5, 445), jnp.float32),
    )(h80, Wl, bl)

    return out2d.reshape(-1, 89, 5), hidden


# speedup vs baseline: 1.8364x; 1.1804x over previous
import functools

import jax
import jax.numpy as jnp
from jax import lax
from jax.experimental import pallas as pl
from jax.experimental.pallas import tpu as pltpu
from jax.experimental.pallas import tpu_sc as plsc

N_NODES = 100000
N_EDGES = 6400000

NC = 2
NS = 16

CHUNK = 128
ROWS = N_EDGES // CHUNK
ROWS_W = 1560
TAIL0 = 32 * ROWS_W
SROWS_D = 120
NSTAGE_D = ROWS_W // SROWS_D
QLAG = 8
SROWS = 40
NSTAGE = ROWS_W // SROWS
RING = 8
PD = 6
NGROUP = SROWS // RING

NPAD = 100352
SLICE = NPAD // NS
ZROWS = 128

_mesh = plsc.VectorSubcoreMesh(
    core_axis_name="c", subcore_axis_name="s", num_cores=NC, num_subcores=NS)


def _deg_body(ei_hbm, deg_out, deg_sh, ids, ones_v, zbuf, qsem):
    c = lax.axis_index("c")
    s = lax.axis_index("s")

    def _zb(i, _):
        zbuf[pl.ds(i * 16, 16)] = jnp.zeros((16,), jnp.float32)
        return 0
    lax.fori_loop(0, SLICE // 16, _zb, 0)

    def _ob(i, _):
        ones_v[pl.ds(i * 16, 16)] = jnp.ones((16,), jnp.float32)
        return 0
    lax.fori_loop(0, 8, _ob, 0)

    pltpu.sync_copy(zbuf, deg_sh.at[pl.ds(s * SLICE, SLICE)])
    plsc.subcore_barrier()

    wid = c * NS + s
    row0 = wid * ROWS_W

    def _wait_one():
        pltpu.make_async_copy(
            ones_v, deg_sh.at[ids.at[0]], qsem).wait()

    def _stage(st, _):
        pltpu.sync_copy(
            ei_hbm.at[1, pl.ds(row0 + st * SROWS_D, SROWS_D)], ids)

        for k in range(QLAG):
            pltpu.async_copy(ones_v, deg_sh.at[ids.at[k]], qsem, add=True)

        def _chunk(j, _):
            _wait_one()
            pltpu.async_copy(
                ones_v, deg_sh.at[ids.at[j + QLAG]], qsem, add=True)
            return 0
        lax.fori_loop(0, SROWS_D - QLAG, _chunk, 0)
        for k in range(QLAG):
            _wait_one()
        return 0
    lax.fori_loop(0, NSTAGE_D, _stage, 0)

    @pl.when(wid < 10)
    def _tail():
        pltpu.sync_copy(ei_hbm.at[1, pl.ds(TAIL0 + wid * 8, 8)],
                        ids.at[pl.ds(0, 8)])
        for k in range(8):
            pltpu.async_copy(ones_v, deg_sh.at[ids.at[k]], qsem, add=True)
        for k in range(8):
            _wait_one()

    plsc.subcore_barrier()
    pltpu.sync_copy(deg_sh.at[pl.ds(s * SLICE, SLICE)], zbuf)
    pltpu.sync_copy(zbuf, deg_out.at[c, pl.ds(s * SLICE, SLICE)])


_deg_call = functools.partial(
    pl.kernel,
    out_type=jax.ShapeDtypeStruct((NC, NPAD), jnp.float32),
    mesh=_mesh,
    scratch_types=[
        pltpu.VMEM_SHARED((NPAD,), jnp.float32),
        pltpu.VMEM((SROWS_D, CHUNK), jnp.int32),
        pltpu.VMEM((CHUNK,), jnp.float32),
        pltpu.VMEM((SLICE,), jnp.float32),
        pltpu.SemaphoreType.DMA,
    ],
    compiler_params=pltpu.CompilerParams(use_tc_tiling_on_sc=False),
)(_deg_body)


def _msg_body(y_hbm, ei_hbm, acc_out, acc_sh, sid, did, rows, zbuf,
              *sems):
    gsems = sems[:RING]
    ssems = sems[RING:]
    c = lax.axis_index("c")
    s = lax.axis_index("s")

    def _zb(i, _):
        zbuf[i, :] = jnp.zeros((16,), jnp.float32)
        return 0
    lax.fori_loop(0, ZROWS, _zb, 0)
    for k in range(SLICE // ZROWS):
        pltpu.sync_copy(zbuf, acc_sh.at[pl.ds(s * SLICE + k * ZROWS, ZROWS)])
    plsc.subcore_barrier()

    wid = c * NS + s
    row0 = wid * ROWS_W

    def _gather(j, b):
        pltpu.async_copy(y_hbm.at[sid.at[j]], rows.at[b], gsems[b])

    def _gather_wait(j, b):
        pltpu.make_async_copy(y_hbm.at[sid.at[j]], rows.at[b],
                              gsems[b]).wait()

    def _scatter(j, b):
        pltpu.async_copy(rows.at[b], acc_sh.at[did.at[j]], ssems[b],
                         add=True)

    def _scatter_wait(j, b):
        pltpu.make_async_copy(rows.at[b], acc_sh.at[did.at[j]],
                              ssems[b]).wait()

    def _stage(st, _):
        base = row0 + st * SROWS
        pltpu.sync_copy(ei_hbm.at[0, pl.ds(base, SROWS)], sid)
        pltpu.sync_copy(ei_hbm.at[1, pl.ds(base, SROWS)], did)

        for b in range(PD):
            _gather(b, b)
        for b in range(RING):
            _gather_wait(b, b)
            _scatter(b, b)
            bp = (b + PD) % RING
            if b + PD >= RING:
                _scatter_wait(b + PD - RING, bp)
            _gather(b + PD, bp)

        def _group(g, _):
            j0 = g * RING
            for b in range(RING):
                j = j0 + b
                _gather_wait(j, b)
                _scatter(j, b)
                bp = (b + PD) % RING
                _scatter_wait(j + PD - RING, bp)
                _gather(j + PD, bp)
            return 0
        lax.fori_loop(1, NGROUP - 1, _group, 0)

        j0 = (NGROUP - 1) * RING
        for b in range(RING):
            j = j0 + b
            _gather_wait(j, b)
            _scatter(j, b)
            if b < RING - PD:
                bp = (b + PD) % RING
                _scatter_wait(j + PD - RING, bp)
                _gather(j + PD, bp)
        for b in range(RING):
            _scatter_wait(j0 + b, b)
        return 0
    lax.fori_loop(0, NSTAGE, _stage, 0)

    @pl.when(wid < 10)
    def _tail():
        pltpu.sync_copy(ei_hbm.at[0, pl.ds(TAIL0 + wid * 8, 8)],
                        sid.at[pl.ds(0, 8)])
        pltpu.sync_copy(ei_hbm.at[1, pl.ds(TAIL0 + wid * 8, 8)],
                        did.at[pl.ds(0, 8)])
        for k in range(8):
            b = k % RING
            pltpu.async_copy(y_hbm.at[sid.at[k]], rows.at[b],
                             gsems[b]).wait()
            pltpu.sync_copy(rows.at[b], acc_sh.at[did.at[k]], add=True)

    plsc.subcore_barrier()
    for k in range(SLICE // ZROWS):
        off = s * SLICE + k * ZROWS
        pltpu.sync_copy(acc_sh.at[pl.ds(off, ZROWS)], zbuf)
        pltpu.sync_copy(zbuf, acc_out.at[c, pl.ds(off, ZROWS)])


_msg_call = functools.partial(
    pl.kernel,
    out_type=jax.ShapeDtypeStruct((NC, NPAD, 16), jnp.float32),
    mesh=_mesh,
    scratch_types=[
        pltpu.VMEM_SHARED((NPAD, 16), jnp.float32),
        pltpu.VMEM((SROWS, CHUNK), jnp.int32),
        pltpu.VMEM((SROWS, CHUNK), jnp.int32),
        pltpu.VMEM((RING, CHUNK, 16), jnp.float32),
        pltpu.VMEM((ZROWS, 16), jnp.float32),
    ] + [pltpu.SemaphoreType.DMA] * (2 * RING),
    compiler_params=pltpu.CompilerParams(use_tc_tiling_on_sc=False),
)(_msg_body)


def _y_kernel(x_ref, w_ref, degs_ref, y_ref, dinv_ref):
    deg = degs_ref[0, :] + degs_ref[1, :] + 1.0
    dinv = lax.rsqrt(deg)[:, None]
    xw = jnp.dot(x_ref[...], w_ref[...], preferred_element_type=jnp.float32)
    y_ref[...] = xw * dinv
    dinv_ref[...] = jnp.broadcast_to(dinv, (dinv.shape[0], 16))


def _combine_kernel(acc_ref, y_ref, dinv_ref, b_ref, wl_ref, bl_ref,
                    hid_ref, out_ref):
    tot = acc_ref[0] + acc_ref[1] + y_ref[...]
    hid = tot * dinv_ref[...] + b_ref[...][None, :]
    hid_ref[...] = hid
    h3 = jnp.maximum(hid, 0.0).reshape(hid.shape[0] // 5, 5, 16)
    o = jnp.dot(h3[:, 0, :], wl_ref[0:16, :],
                preferred_element_type=jnp.float32)
    for k in range(1, 5):
        o = o + jnp.dot(h3[:, k, :], wl_ref[16 * k:16 * (k + 1), :],
                        preferred_element_type=jnp.float32)
    out_ref[...] = o + bl_ref[...][None, :]


def kernel(x, edge_index, W, b, Wl, bl):
    ei3 = edge_index.reshape(2, ROWS, CHUNK)

    degs = _deg_call(ei3)

    y, dinv16 = pl.pallas_call(
        _y_kernel,
        grid=(NS,),
        in_specs=[
            pl.BlockSpec((SLICE, 8), lambda i: (i, 0)),
            pl.BlockSpec((8, 16), lambda i: (0, 0)),
            pl.BlockSpec((NC, SLICE), lambda i: (0, i)),
        ],
        out_specs=[
            pl.BlockSpec((SLICE, 16), lambda i: (i, 0)),
            pl.BlockSpec((SLICE, 16), lambda i: (i, 0)),
        ],
        out_shape=[
            jax.ShapeDtypeStruct((NPAD, 16), jnp.float32),
            jax.ShapeDtypeStruct((NPAD, 16), jnp.float32),
        ],
    )(x, W, degs)

    accs = _msg_call(y, ei3)

    BR = 4000
    hidden, out2d = pl.pallas_call(
        _combine_kernel,
        grid=(N_NODES // BR,),
        in_specs=[
            pl.BlockSpec((NC, BR, 16), lambda i: (0, i, 0)),
            pl.BlockSpec((BR, 16), lambda i: (i, 0)),
            pl.BlockSpec((BR, 16), lambda i: (i, 0)),
            pl.BlockSpec((16,), lambda i: (0,)),
            pl.BlockSpec((80, 445), lambda i: (0, 0)),
            pl.BlockSpec((445,), lambda i: (0,)),
        ],
        out_specs=[
            pl.BlockSpec((BR, 16), lambda i: (i, 0)),
            pl.BlockSpec((BR // 5, 445), lambda i: (i, 0)),
        ],
        out_shape=[
            jax.ShapeDtypeStruct((N_NODES, 16), jnp.float32),
            jax.ShapeDtypeStruct((N_NODES // 5, 445), jnp.float32),
        ],
    )(accs, y, dinv16, b, Wl, bl)

    return out2d.reshape(-1, 89, 5), hidden


# speedup vs baseline: 1.8835x; 1.0257x over previous
import functools

import jax
import jax.numpy as jnp
from jax import lax
from jax.experimental import pallas as pl
from jax.experimental.pallas import tpu as pltpu
from jax.experimental.pallas import tpu_sc as plsc

N_NODES = 100000
N_EDGES = 6400000

NC = 2
NS = 16

CHUNK = 128
ROWS = N_EDGES // CHUNK
ROWS_W = 1560
TAIL0 = 32 * ROWS_W
SROWS_D = 120
NSTAGE_D = ROWS_W // SROWS_D
QLAG = 8
SROWS = 40
NSTAGE = ROWS_W // SROWS
RING = 8
PD = 6
NGROUP = SROWS // RING

NPAD = 100352
SLICE = NPAD // NS
ZROWS = 128

_mesh = plsc.VectorSubcoreMesh(
    core_axis_name="c", subcore_axis_name="s", num_cores=NC, num_subcores=NS)


def _deg_body(ei_hbm, deg_out, deg_sh, ids, ones_v, zbuf, qsem):
    c = lax.axis_index("c")
    s = lax.axis_index("s")

    def _zb(i, _):
        zbuf[pl.ds(i * 16, 16)] = jnp.zeros((16,), jnp.float32)
        return 0
    lax.fori_loop(0, SLICE // 16, _zb, 0)

    def _ob(i, _):
        ones_v[pl.ds(i * 16, 16)] = jnp.ones((16,), jnp.float32)
        return 0
    lax.fori_loop(0, 8, _ob, 0)

    pltpu.sync_copy(zbuf, deg_sh.at[pl.ds(s * SLICE, SLICE)])
    plsc.subcore_barrier()

    wid = c * NS + s
    row0 = wid * ROWS_W

    def _wait_one():
        pltpu.make_async_copy(
            ones_v, deg_sh.at[ids.at[0]], qsem).wait()

    def _stage(st, _):
        pltpu.sync_copy(
            ei_hbm.at[1, pl.ds(row0 + st * SROWS_D, SROWS_D)], ids)

        for k in range(QLAG):
            pltpu.async_copy(ones_v, deg_sh.at[ids.at[k]], qsem, add=True)

        def _chunk(j, _):
            _wait_one()
            pltpu.async_copy(
                ones_v, deg_sh.at[ids.at[j + QLAG]], qsem, add=True)
            return 0
        lax.fori_loop(0, SROWS_D - QLAG, _chunk, 0)
        for k in range(QLAG):
            _wait_one()
        return 0
    lax.fori_loop(0, NSTAGE_D, _stage, 0)

    @pl.when(wid < 10)
    def _tail():
        pltpu.sync_copy(ei_hbm.at[1, pl.ds(TAIL0 + wid * 8, 8)],
                        ids.at[pl.ds(0, 8)])
        for k in range(8):
            pltpu.async_copy(ones_v, deg_sh.at[ids.at[k]], qsem, add=True)
        for k in range(8):
            _wait_one()

    plsc.subcore_barrier()
    pltpu.sync_copy(deg_sh.at[pl.ds(s * SLICE, SLICE)], zbuf)
    pltpu.sync_copy(zbuf, deg_out.at[c, pl.ds(s * SLICE, SLICE)])


_deg_call = functools.partial(
    pl.kernel,
    out_type=jax.ShapeDtypeStruct((NC, NPAD), jnp.float32),
    mesh=_mesh,
    scratch_types=[
        pltpu.VMEM_SHARED((NPAD,), jnp.float32),
        pltpu.VMEM((SROWS_D, CHUNK), jnp.int32),
        pltpu.VMEM((CHUNK,), jnp.float32),
        pltpu.VMEM((SLICE,), jnp.float32),
        pltpu.SemaphoreType.DMA,
    ],
    compiler_params=pltpu.CompilerParams(use_tc_tiling_on_sc=False),
)(_deg_body)


def _msg_body(y_hbm, ei_hbm, acc_out, acc_sh, sid, did, rows, zbuf, stsem,
              *sems):
    gsems = sems[:RING]
    ssems = sems[RING:]
    c = lax.axis_index("c")
    s = lax.axis_index("s")

    def _zb(i, _):
        zbuf[i, :] = jnp.zeros((16,), jnp.float32)
        return 0
    lax.fori_loop(0, ZROWS, _zb, 0)
    for k in range(SLICE // ZROWS):
        pltpu.sync_copy(zbuf, acc_sh.at[pl.ds(s * SLICE + k * ZROWS, ZROWS)])
    plsc.subcore_barrier()

    wid = c * NS + s
    row0 = wid * ROWS_W

    def _gather(j, b):
        pltpu.async_copy(y_hbm.at[sid.at[j]], rows.at[b], gsems[b])

    def _gather_wait(j, b):
        pltpu.make_async_copy(y_hbm.at[sid.at[j]], rows.at[b],
                              gsems[b]).wait()

    def _scatter(j, b):
        pltpu.async_copy(rows.at[b], acc_sh.at[did.at[j]], ssems[b],
                         add=True)

    def _scatter_wait(j, b):
        pltpu.make_async_copy(rows.at[b], acc_sh.at[did.at[j]],
                              ssems[b]).wait()

    def _stage(st, _):
        base = row0 + st * SROWS
        d0 = pltpu.async_copy(ei_hbm.at[0, pl.ds(base, SROWS)], sid, stsem)
        d1 = pltpu.async_copy(ei_hbm.at[1, pl.ds(base, SROWS)], did, stsem)
        d0.wait()
        d1.wait()

        for b in range(PD):
            _gather(b, b)
        for b in range(RING):
            _gather_wait(b, b)
            _scatter(b, b)
            bp = (b + PD) % RING
            if b + PD >= RING:
                _scatter_wait(b + PD - RING, bp)
            _gather(b + PD, bp)

        def _group(g, _):
            j0 = g * RING
            for b in range(RING):
                j = j0 + b
                _gather_wait(j, b)
                _scatter(j, b)
                bp = (b + PD) % RING
                _scatter_wait(j + PD - RING, bp)
                _gather(j + PD, bp)
            return 0
        lax.fori_loop(1, NGROUP - 1, _group, 0)

        j0 = (NGROUP - 1) * RING
        for b in range(RING):
            j = j0 + b
            _gather_wait(j, b)
            _scatter(j, b)
            if b < RING - PD:
                bp = (b + PD) % RING
                _scatter_wait(j + PD - RING, bp)
                _gather(j + PD, bp)
        for b in range(RING):
            _scatter_wait(j0 + b, b)
        return 0
    lax.fori_loop(0, NSTAGE, _stage, 0)

    @pl.when(wid < 10)
    def _tail():
        pltpu.sync_copy(ei_hbm.at[0, pl.ds(TAIL0 + wid * 8, 8)],
                        sid.at[pl.ds(0, 8)])
        pltpu.sync_copy(ei_hbm.at[1, pl.ds(TAIL0 + wid * 8, 8)],
                        did.at[pl.ds(0, 8)])
        for k in range(8):
            b = k % RING
            pltpu.async_copy(y_hbm.at[sid.at[k]], rows.at[b],
                             gsems[b]).wait()
            pltpu.sync_copy(rows.at[b], acc_sh.at[did.at[k]], add=True)

    plsc.subcore_barrier()
    for k in range(SLICE // ZROWS):
        off = s * SLICE + k * ZROWS
        pltpu.sync_copy(acc_sh.at[pl.ds(off, ZROWS)], zbuf)
        pltpu.sync_copy(zbuf, acc_out.at[c, pl.ds(off, ZROWS)])


_msg_call = functools.partial(
    pl.kernel,
    out_type=jax.ShapeDtypeStruct((NC, NPAD, 16), jnp.float32),
    mesh=_mesh,
    scratch_types=[
        pltpu.VMEM_SHARED((NPAD, 16), jnp.float32),
        pltpu.VMEM((SROWS, CHUNK), jnp.int32),
        pltpu.VMEM((SROWS, CHUNK), jnp.int32),
        pltpu.VMEM((RING, CHUNK, 16), jnp.float32),
        pltpu.VMEM((ZROWS, 16), jnp.float32),
        pltpu.SemaphoreType.DMA,
    ] + [pltpu.SemaphoreType.DMA] * (2 * RING),
    compiler_params=pltpu.CompilerParams(use_tc_tiling_on_sc=False),
)(_msg_body)


def _y_kernel(x_ref, w_ref, degs_ref, y_ref, dinv_ref):
    deg = degs_ref[0, :] + degs_ref[1, :] + 1.0
    dinv = lax.rsqrt(deg)[:, None]
    xw = jnp.dot(x_ref[...], w_ref[...], preferred_element_type=jnp.float32)
    y_ref[...] = xw * dinv
    dinv_ref[...] = jnp.broadcast_to(dinv, (dinv.shape[0], 16))


def _combine_kernel(acc_ref, y_ref, dinv_ref, b_ref, wl_ref, bl_ref,
                    hid_ref, out_ref):
    tot = acc_ref[0] + acc_ref[1] + y_ref[...]
    hid = tot * dinv_ref[...] + b_ref[...][None, :]
    hid_ref[...] = hid
    h3 = jnp.maximum(hid, 0.0).reshape(hid.shape[0] // 5, 5, 16)
    o = jnp.dot(h3[:, 0, :], wl_ref[0:16, :],
                preferred_element_type=jnp.float32)
    for k in range(1, 5):
        o = o + jnp.dot(h3[:, k, :], wl_ref[16 * k:16 * (k + 1), :],
                        preferred_element_type=jnp.float32)
    out_ref[...] = o + bl_ref[...][None, :]


def kernel(x, edge_index, W, b, Wl, bl):
    ei3 = edge_index.reshape(2, ROWS, CHUNK)

    degs = _deg_call(ei3)

    y, dinv16 = pl.pallas_call(
        _y_kernel,
        grid=(NS,),
        in_specs=[
            pl.BlockSpec((SLICE, 8), lambda i: (i, 0)),
            pl.BlockSpec((8, 16), lambda i: (0, 0)),
            pl.BlockSpec((NC, SLICE), lambda i: (0, i)),
        ],
        out_specs=[
            pl.BlockSpec((SLICE, 16), lambda i: (i, 0)),
            pl.BlockSpec((SLICE, 16), lambda i: (i, 0)),
        ],
        out_shape=[
            jax.ShapeDtypeStruct((NPAD, 16), jnp.float32),
            jax.ShapeDtypeStruct((NPAD, 16), jnp.float32),
        ],
    )(x, W, degs)

    accs = _msg_call(y, ei3)

    BR = 4000
    hidden, out2d = pl.pallas_call(
        _combine_kernel,
        grid=(N_NODES // BR,),
        in_specs=[
            pl.BlockSpec((NC, BR, 16), lambda i: (0, i, 0)),
            pl.BlockSpec((BR, 16), lambda i: (i, 0)),
            pl.BlockSpec((BR, 16), lambda i: (i, 0)),
            pl.BlockSpec((16,), lambda i: (0,)),
            pl.BlockSpec((80, 445), lambda i: (0, 0)),
            pl.BlockSpec((445,), lambda i: (0,)),
        ],
        out_specs=[
            pl.BlockSpec((BR, 16), lambda i: (i, 0)),
            pl.BlockSpec((BR // 5, 445), lambda i: (i, 0)),
        ],
        out_shape=[
            jax.ShapeDtypeStruct((N_NODES, 16), jnp.float32),
            jax.ShapeDtypeStruct((N_NODES // 5, 445), jnp.float32),
        ],
    )(accs, y, dinv16, b, Wl, bl)

    return out2d.reshape(-1, 89, 5), hidden
